# baseline (speedup 1.0000x reference)
import functools

import jax
import jax.numpy as jnp
from jax import lax
from jax.experimental import pallas as pl
from jax.experimental.pallas import tpu as pltpu
from jax.experimental.pallas import tpu_sc as plsc

_R, _N = 128, 8192
_NC, _NS = 2, 16
_NW = _NC * _NS
_ROWS_PER_W = _R // _NW
_LANES = 16


def _season_body(x_ref, season_ref):
    xv = x_ref[...]
    row0 = (jax.lax.broadcasted_iota(jnp.int32, xv.shape, 0) == 0) & (
        pl.program_id(0) == 0
    )
    season_ref[...] = jnp.where(row0, jnp.zeros((), xv.dtype), xv)


def _season_tc(x):
    n, m = x.shape
    blk = 64
    spec = pl.BlockSpec((blk, m), lambda i: (i, 0))
    return pl.pallas_call(
        _season_body,
        grid=(n // blk,),
        in_specs=[spec],
        out_specs=spec,
        out_shape=jax.ShapeDtypeStruct((n, m), x.dtype),
    )(x)


def _trend_body(x_hbm, out_hbm, zrow, xrow, sem):
    wid = lax.axis_index("s") * _NC + lax.axis_index("c")
    base = wid * _ROWS_PER_W
    zero16 = jnp.zeros((_LANES,), jnp.float32)

    def _zfill(i, carry):
        zrow[pl.ds(i * _LANES, _LANES)] = zero16
        return carry

    lax.fori_loop(0, _N // _LANES, _zfill, 0)

    copies = [
        pltpu.async_copy(zrow, out_hbm.at[base + j], sem)
        for j in range(_ROWS_PER_W)
    ]
    for c in copies:
        c.wait()

    @pl.when(wid == 0)
    def _():
        pltpu.sync_copy(x_hbm.at[0], xrow)
        pltpu.sync_copy(xrow, out_hbm.at[0])


_trend_sc = functools.partial(
    pl.kernel,
    mesh=plsc.VectorSubcoreMesh(core_axis_name="c", subcore_axis_name="s"),
    out_type=jax.ShapeDtypeStruct((_R, _N), jnp.float32),
    scratch_types=[
        pltpu.VMEM((_N,), jnp.float32),
        pltpu.VMEM((_N,), jnp.float32),
        pltpu.SemaphoreType.DMA,
    ],
)(_trend_body)


def kernel(x):
    season = _season_tc(x)
    trend = _trend_sc(x)
    return (season, trend)


# speedup vs baseline: 42.3846x; 42.3846x over previous
import jax
import jax.numpy as jnp
from jax.experimental import pallas as pl


def _decomp_body(x_ref, season_ref, trend_ref):
    xv = x_ref[...]
    row0 = jax.lax.broadcasted_iota(jnp.int32, xv.shape, 0) == 0
    zero = jnp.zeros((), xv.dtype)
    season_ref[...] = jnp.where(row0, zero, xv)
    trend_ref[...] = jnp.where(row0, xv, zero)


def kernel(x):
    n, m = x.shape
    blk = m // 2
    spec = pl.BlockSpec((n, blk), lambda i: (0, i))
    season, trend = pl.pallas_call(
        _decomp_body,
        grid=(n // blk,),
        in_specs=[spec],
        out_specs=(spec, spec),
        out_shape=(
            jax.ShapeDtypeStruct((n, m), x.dtype),
            jax.ShapeDtypeStruct((n, m), x.dtype),
        ),
    )(x)
    return (season, trend)
